# speedup vs baseline: 1.7586x; 1.7586x over previous
import functools

import jax
import jax.numpy as jnp
from jax.experimental import pallas as pl

_INTERPRET = False


def _router_kernel(x_ref, wg_ref, w_ref):
    x = x_ref[...]
    logits = jax.lax.dot_general(
        x, wg_ref[...], (((1,), (1,)), ((), ())),
        preferred_element_type=jnp.float32)
    e_iota = jax.lax.broadcasted_iota(jnp.int32, logits.shape, 1)
    i1 = jnp.argmax(logits, axis=1)[:, None]
    m1 = jnp.max(logits, axis=1, keepdims=True)
    masked = jnp.where(e_iota == i1, -jnp.inf, logits)
    i2 = jnp.argmax(masked, axis=1)[:, None]
    m2 = jnp.max(masked, axis=1, keepdims=True)
    e2 = jnp.exp(m2 - m1)
    p1 = 1.0 / (1.0 + e2)
    p2 = e2 / (1.0 + e2)
    w_ref[...] = jnp.where(e_iota == i1, p1, 0.0) + jnp.where(e_iota == i2, p2, 0.0)


def _moe_kernel(w_ref, x_ref, w1_ref, w2_ref, wp_ref, y_ref):
    e = pl.program_id(0)
    i = pl.program_id(1)

    @pl.when((e == 0) & (i == 0))
    def _init():
        y_ref[...] = jnp.zeros_like(y_ref)

    x = x_ref[...].astype(jnp.bfloat16)
    w1 = w1_ref[0].astype(jnp.bfloat16)
    w2 = w2_ref[0].astype(jnp.bfloat16)
    wp = wp_ref[0].astype(jnp.bfloat16)
    h1 = jax.lax.dot_general(x, w1, (((1,), (1,)), ((), ())),
                             preferred_element_type=jnp.float32)
    h2 = jax.lax.dot_general(x, w2, (((1,), (1,)), ((), ())),
                             preferred_element_type=jnp.float32)
    h = (h1 * jax.nn.sigmoid(h1) * h2).astype(jnp.bfloat16)
    out = jax.lax.dot_general(h, wp, (((1,), (1,)), ((), ())),
                              preferred_element_type=jnp.float32)
    e_iota = jax.lax.broadcasted_iota(jnp.int32, w_ref.shape, 1)
    wcol = jnp.sum(w_ref[...] * (e_iota == e).astype(jnp.float32),
                   axis=1, keepdims=True)
    y_ref[...] += wcol * out


def kernel(x, Wg, W1, W2, Wp):
    Bv, Tv, C = x.shape
    E, I, _ = W1.shape
    N = Bv * Tv
    xf = x.reshape(N, C)

    w = pl.pallas_call(
        _router_kernel,
        out_shape=jax.ShapeDtypeStruct((N, E), jnp.float32),
        interpret=_INTERPRET,
    )(xf, Wg)

    i_blk = 256 if I % 256 == 0 else I
    n_iblk = I // i_blk

    y = pl.pallas_call(
        _moe_kernel,
        grid=(E, n_iblk),
        in_specs=[
            pl.BlockSpec((N, E), lambda e, i: (0, 0)),
            pl.BlockSpec((N, C), lambda e, i: (0, 0)),
            pl.BlockSpec((1, i_blk, C), lambda e, i: (e, i, 0)),
            pl.BlockSpec((1, i_blk, C), lambda e, i: (e, i, 0)),
            pl.BlockSpec((1, C, i_blk), lambda e, i: (e, 0, i)),
        ],
        out_specs=pl.BlockSpec((N, C), lambda e, i: (0, 0)),
        out_shape=jax.ShapeDtypeStruct((N, C), jnp.float32),
        interpret=_INTERPRET,
    )(w, xf, W1, W2, Wp)

    return y.reshape(Bv, Tv, C)
